# baseline (speedup 1.0000x reference)
import functools

import jax
import jax.numpy as jnp
from jax import lax
from jax.experimental import pallas as pl
from jax.experimental.pallas import tpu as pltpu
from jax.experimental.pallas import tpu_sc as plsc

N = 10000
D = 256
DH = 128
E = 160000
NCORE = 2
NSUB = 16
EB = 128
NBT = 1280
CH8 = 8
NCHUNK = NBT // CH8
SROWS = 632
ACCR = 640
TROWS = NSUB * SROWS
DUMPDST = 4 * TROWS
BN = 1000


_INV_SQRT2 = 0.7071067811865476


def _gelu(h):
    return 0.5 * h * (1.0 + lax.erf(h * _INV_SQRT2))


def _mm0_body(x_ref, w_ref, b_ref, o_ref):
    r = jnp.dot(x_ref[...], w_ref[...], preferred_element_type=jnp.float32)
    r = r + b_ref[...]
    o_ref[0] = r[:, :DH]
    o_ref[1] = r[:, DH:]


def _fused_body(a_ref, g_ref, be_ref, w_ref, b_ref, o_ref):
    a = jnp.concatenate([a_ref[0], a_ref[1]], axis=-1)
    mu = jnp.mean(a, axis=-1, keepdims=True)
    var = jnp.mean((a - mu) ** 2, axis=-1, keepdims=True)
    h = (a - mu) * lax.rsqrt(var + 1e-5) * g_ref[...] + be_ref[...]
    h = _gelu(h)
    r = jnp.dot(h, w_ref[...], preferred_element_type=jnp.float32)
    r = r + b_ref[...]
    o_ref[0] = r[:, :DH]
    o_ref[1] = r[:, DH:]


def _final_body(a_ref, g_ref, be_ref, w_ref, b_ref, o_ref):
    a = jnp.concatenate([a_ref[0], a_ref[1]], axis=-1)
    mu = jnp.mean(a, axis=-1, keepdims=True)
    var = jnp.mean((a - mu) ** 2, axis=-1, keepdims=True)
    h = (a - mu) * lax.rsqrt(var + 1e-5) * g_ref[...] + be_ref[...]
    h = _gelu(h)
    r = jnp.dot(h, w_ref[...], preferred_element_type=jnp.float32)
    o_ref[...] = r + b_ref[...]


def _mm0(x, W, b):
    return pl.pallas_call(
        _mm0_body,
        grid=(N // BN,),
        in_specs=[
            pl.BlockSpec((BN, D), lambda i: (i, 0)),
            pl.BlockSpec((D, D), lambda i: (0, 0)),
            pl.BlockSpec((1, D), lambda i: (0, 0)),
        ],
        out_specs=pl.BlockSpec((2, BN, DH), lambda i: (0, i, 0)),
        out_shape=jax.ShapeDtypeStruct((2, N, DH), jnp.float32),
    )(x, W, b.reshape(1, D))


def _fused(a, g, be, W, b):
    return pl.pallas_call(
        _fused_body,
        grid=(N // BN,),
        in_specs=[
            pl.BlockSpec((2, BN, DH), lambda i: (0, i, 0)),
            pl.BlockSpec((1, D), lambda i: (0, 0)),
            pl.BlockSpec((1, D), lambda i: (0, 0)),
            pl.BlockSpec((D, D), lambda i: (0, 0)),
            pl.BlockSpec((1, D), lambda i: (0, 0)),
        ],
        out_specs=pl.BlockSpec((2, BN, DH), lambda i: (0, i, 0)),
        out_shape=jax.ShapeDtypeStruct((2, N, DH), jnp.float32),
    )(a, g.reshape(1, D), be.reshape(1, D), W, b.reshape(1, D))


def _final(a, g, be, W, b):
    return pl.pallas_call(
        _final_body,
        grid=(N // BN,),
        in_specs=[
            pl.BlockSpec((2, BN, DH), lambda i: (0, i, 0)),
            pl.BlockSpec((1, D), lambda i: (0, 0)),
            pl.BlockSpec((1, D), lambda i: (0, 0)),
            pl.BlockSpec((D, D), lambda i: (0, 0)),
            pl.BlockSpec((1, D), lambda i: (0, 0)),
        ],
        out_specs=pl.BlockSpec((BN, D), lambda i: (i, 0)),
        out_shape=jax.ShapeDtypeStruct((N, D), jnp.float32),
    )(a, g.reshape(1, D), be.reshape(1, D), W, b.reshape(1, D))


_SC_MESH = plsc.VectorSubcoreMesh(
    core_axis_name="c", subcore_axis_name="s", num_cores=NCORE, num_subcores=NSUB
)


@functools.partial(
    pl.kernel,
    out_type=jax.ShapeDtypeStruct((NCORE, TROWS, DH), jnp.float32),
    mesh=_SC_MESH,
    scratch_types=[
        pltpu.VMEM((ACCR, DH), jnp.float32),
        pltpu.VMEM((CH8, EB), jnp.int32),
        pltpu.VMEM((CH8, EB), jnp.int32),
        pltpu.VMEM((2, EB, DH), jnp.float32),
        pltpu.VMEM((NSUB, 16), jnp.int32),
        pltpu.SemaphoreType.DMA,
    ],
)
def _sc_scatter(hlin_hbm, srcg_hbm, dstg_hbm, zeros_hbm, ckb_hbm, out_hbm,
                acc, src_v, dst_v, rows_v, bounds_v, gsem):
    cid = lax.axis_index("c")
    sid = lax.axis_index("s")

    pltpu.sync_copy(ckb_hbm, bounds_v)
    pltpu.sync_copy(zeros_hbm, acc)
    brow = bounds_v[sid, pl.ds(0, 16)]
    klo = brow[0]
    khi = brow[1]
    base = sid * SROWS

    def chunk_body(k, carry):
        cb = k * CH8
        pltpu.sync_copy(srcg_hbm.at[pl.ds(cb, CH8)], src_v)
        pltpu.sync_copy(dstg_hbm.at[pl.ds(cb, CH8)], dst_v)
        pltpu.async_copy(hlin_hbm.at[cid].at[src_v.at[0]], rows_v.at[0], gsem)

        def b_body(j, c3):
            nxt = j + 1

            @pl.when(nxt < CH8)
            def _():
                pltpu.async_copy(hlin_hbm.at[cid].at[src_v.at[nxt]],
                                 rows_v.at[nxt % 2], gsem)

            pltpu.make_async_copy(hlin_hbm.at[cid].at[src_v.at[j]],
                                  rows_v.at[j % 2], gsem).wait()
            b = j % 2

            def g_body(g, c4):
                lv = dst_v[j, pl.ds(g * 16, 16)]
                l0 = lv - base
                oob = (l0 < 0) | (l0 >= SROWS)
                loc = jnp.where(oob, SROWS, l0)
                for t in range(16):
                    row = loc[t]
                    e = g * 16 + t
                    for c in range(8):
                        v = rows_v[b, e, pl.ds(c * 16, 16)]
                        plsc.addupdate(acc.at[row, pl.ds(c * 16, 16)], v)
                return c4

            lax.fori_loop(0, 8, g_body, 0)
            return c3

        lax.fori_loop(0, CH8, b_body, 0)
        return carry

    lax.fori_loop(klo, khi, chunk_body, 0)

    pltpu.sync_copy(acc.at[pl.ds(0, SROWS)],
                    out_hbm.at[cid].at[pl.ds(sid * SROWS, SROWS)])


def kernel(x, edge_index, W0, b0, g0, be0, W1, b1, g1, be1,
           W2, b2, g2, be2, W3, b3, g3, be3, Wo, bo):
    src = edge_index[0]
    dst = edge_index[1]
    order = jnp.argsort(dst)
    src = src[order]
    dst = dst[order]
    pad = NBT * EB - E
    srcg = jnp.concatenate(
        [src, jnp.zeros((pad,), jnp.int32)]).reshape(NBT, EB)
    dstp = jnp.concatenate([dst, jnp.full((pad,), DUMPDST, jnp.int32)])
    dstg = dstp.reshape(NBT, EB)
    bases = (jnp.arange(NSUB + 1, dtype=jnp.int32) * SROWS).astype(jnp.int32)
    starts = jnp.searchsorted(dstp, bases).astype(jnp.int32)
    per = EB * CH8
    ck_lo = starts[:NSUB] // per
    ck_hi = (starts[1:] + per - 1) // per
    ckb = jnp.stack(
        [ck_lo, ck_hi] + [jnp.zeros((NSUB,), jnp.int32)] * 14,
        axis=1).astype(jnp.int32)
    zeros = jnp.zeros((ACCR, DH), jnp.float32)

    hlin = _mm0(x, W0, b0)
    layers = [(g0, be0, W1, b1), (g1, be1, W2, b2), (g2, be2, W3, b3)]
    for (g, be, W, b) in layers:
        agg = _sc_scatter(hlin, srcg, dstg, zeros, ckb)
        hlin = _fused(agg, g, be, W, b)
    agg = _sc_scatter(hlin, srcg, dstg, zeros, ckb)
    return _final(agg, g3, be3, Wo, bo)


# speedup vs baseline: 1.3495x; 1.3495x over previous
import functools

import jax
import jax.numpy as jnp
from jax import lax
from jax.experimental import pallas as pl
from jax.experimental.pallas import tpu as pltpu
from jax.experimental.pallas import tpu_sc as plsc

N = 10000
D = 256
DH = 128
E = 160000
NCORE = 2
NSUB = 16
EB = 128
NB = 80
HNB = 40
EPS = NB * EB
EPAD = NSUB * EPS
TROWS = 10112
ZR = TROWS // NSUB
BN = 1000


_INV_SQRT2 = 0.7071067811865476


def _gelu(h):
    return 0.5 * h * (1.0 + lax.erf(h * _INV_SQRT2))


def _mm0_body(x_ref, w_ref, b_ref, o_ref):
    r = jnp.dot(x_ref[...], w_ref[...], preferred_element_type=jnp.float32)
    r = r + b_ref[...]
    o_ref[0] = r[:, :DH]
    o_ref[1] = r[:, DH:]


def _fused_body(a_ref, g_ref, be_ref, w_ref, b_ref, o_ref):
    a = jnp.concatenate([a_ref[0], a_ref[1]], axis=-1)
    mu = jnp.mean(a, axis=-1, keepdims=True)
    var = jnp.mean((a - mu) ** 2, axis=-1, keepdims=True)
    h = (a - mu) * lax.rsqrt(var + 1e-5) * g_ref[...] + be_ref[...]
    h = _gelu(h)
    r = jnp.dot(h, w_ref[...], preferred_element_type=jnp.float32)
    r = r + b_ref[...]
    o_ref[0] = r[:, :DH]
    o_ref[1] = r[:, DH:]


def _final_body(a_ref, g_ref, be_ref, w_ref, b_ref, o_ref):
    a = jnp.concatenate([a_ref[0], a_ref[1]], axis=-1)
    mu = jnp.mean(a, axis=-1, keepdims=True)
    var = jnp.mean((a - mu) ** 2, axis=-1, keepdims=True)
    h = (a - mu) * lax.rsqrt(var + 1e-5) * g_ref[...] + be_ref[...]
    h = _gelu(h)
    r = jnp.dot(h, w_ref[...], preferred_element_type=jnp.float32)
    o_ref[...] = r + b_ref[...]


def _mm0(x, W, b):
    return pl.pallas_call(
        _mm0_body,
        grid=(N // BN,),
        in_specs=[
            pl.BlockSpec((BN, D), lambda i: (i, 0)),
            pl.BlockSpec((D, D), lambda i: (0, 0)),
            pl.BlockSpec((1, D), lambda i: (0, 0)),
        ],
        out_specs=pl.BlockSpec((2, BN, DH), lambda i: (0, i, 0)),
        out_shape=jax.ShapeDtypeStruct((2, N, DH), jnp.float32),
    )(x, W, b.reshape(1, D))


def _fused(a, g, be, W, b):
    return pl.pallas_call(
        _fused_body,
        grid=(N // BN,),
        in_specs=[
            pl.BlockSpec((2, BN, DH), lambda i: (0, i, 0)),
            pl.BlockSpec((1, D), lambda i: (0, 0)),
            pl.BlockSpec((1, D), lambda i: (0, 0)),
            pl.BlockSpec((D, D), lambda i: (0, 0)),
            pl.BlockSpec((1, D), lambda i: (0, 0)),
        ],
        out_specs=pl.BlockSpec((2, BN, DH), lambda i: (0, i, 0)),
        out_shape=jax.ShapeDtypeStruct((2, N, DH), jnp.float32),
    )(a, g.reshape(1, D), be.reshape(1, D), W, b.reshape(1, D))


def _final(a, g, be, W, b):
    return pl.pallas_call(
        _final_body,
        grid=(N // BN,),
        in_specs=[
            pl.BlockSpec((2, BN, DH), lambda i: (0, i, 0)),
            pl.BlockSpec((1, D), lambda i: (0, 0)),
            pl.BlockSpec((1, D), lambda i: (0, 0)),
            pl.BlockSpec((D, D), lambda i: (0, 0)),
            pl.BlockSpec((1, D), lambda i: (0, 0)),
        ],
        out_specs=pl.BlockSpec((BN, D), lambda i: (i, 0)),
        out_shape=jax.ShapeDtypeStruct((N, D), jnp.float32),
    )(a, g.reshape(1, D), be.reshape(1, D), W, b.reshape(1, D))


_SC_MESH = plsc.VectorSubcoreMesh(
    core_axis_name="c", subcore_axis_name="s", num_cores=NCORE, num_subcores=NSUB
)


@functools.partial(
    pl.kernel,
    out_type=jax.ShapeDtypeStruct((NCORE, TROWS, DH), jnp.float32),
    mesh=_SC_MESH,
    scratch_types=[
        pltpu.VMEM_SHARED((TROWS, DH), jnp.float32),
        pltpu.VMEM((HNB, EB), jnp.int32),
        pltpu.VMEM((HNB, EB), jnp.int32),
        pltpu.VMEM((2, EB, DH), jnp.float32),
        pltpu.SemaphoreType.DMA,
        pltpu.SemaphoreType.DMA,
    ],
)
def _sc_scatter(hlin_hbm, srcg_hbm, dstg_hbm, zeros_hbm, out_hbm,
                table, src_v, dst_v, rows_v, gsem, gsem2):

    def _gather(j, buf):
        pltpu.async_copy(hlin_hbm.at[cid].at[src_v.at[j, pl.ds(0, EB // 2)]],
                         rows_v.at[buf].at[pl.ds(0, EB // 2)], gsem)
        pltpu.async_copy(
            hlin_hbm.at[cid].at[src_v.at[j, pl.ds(EB // 2, EB // 2)]],
            rows_v.at[buf].at[pl.ds(EB // 2, EB // 2)], gsem2)

    def _gwait(j, buf):
        pltpu.make_async_copy(
            hlin_hbm.at[cid].at[src_v.at[j, pl.ds(0, EB // 2)]],
            rows_v.at[buf].at[pl.ds(0, EB // 2)], gsem).wait()
        pltpu.make_async_copy(
            hlin_hbm.at[cid].at[src_v.at[j, pl.ds(EB // 2, EB // 2)]],
            rows_v.at[buf].at[pl.ds(EB // 2, EB // 2)], gsem2).wait()
    cid = lax.axis_index("c")
    sid = lax.axis_index("s")

    pltpu.sync_copy(zeros_hbm.at[pl.ds(sid * ZR, ZR)],
                    table.at[pl.ds(sid * ZR, ZR)])
    plsc.subcore_barrier()

    def body(j, carry):
        nxt = j + 1

        @pl.when(nxt < HNB)
        def _():
            _gather(nxt, nxt % 2)

        _gwait(j, j % 2)
        pltpu.sync_copy(rows_v.at[j % 2], table.at[dst_v.at[j]], add=True)
        return carry

    for h in range(NB // HNB):
        pltpu.sync_copy(srcg_hbm.at[sid].at[pl.ds(h * HNB, HNB)], src_v)
        pltpu.sync_copy(dstg_hbm.at[sid].at[pl.ds(h * HNB, HNB)], dst_v)
        _gather(0, 0)
        lax.fori_loop(0, HNB, body, 0)
    plsc.subcore_barrier()

    pltpu.sync_copy(table.at[pl.ds(sid * ZR, ZR)],
                    out_hbm.at[cid].at[pl.ds(sid * ZR, ZR)])


def kernel(x, edge_index, W0, b0, g0, be0, W1, b1, g1, be1,
           W2, b2, g2, be2, W3, b3, g3, be3, Wo, bo):
    src = edge_index[0]
    dst = edge_index[1]
    pad = EPAD - E
    srcg = jnp.concatenate(
        [src, jnp.zeros((pad,), jnp.int32)]).reshape(NSUB, NB, EB)
    dstg = jnp.concatenate(
        [dst, jnp.full((pad,), N, jnp.int32)]).reshape(NSUB, NB, EB)
    zeros = jnp.zeros((TROWS, DH), jnp.float32)

    hlin = _mm0(x, W0, b0)
    layers = [(g0, be0, W1, b1), (g1, be1, W2, b2), (g2, be2, W3, b3)]
    for (g, be, W, b) in layers:
        agg = _sc_scatter(hlin, srcg, dstg, zeros)
        hlin = _fused(agg, g, be, W, b)
    agg = _sc_scatter(hlin, srcg, dstg, zeros)
    return _final(agg, g3, be3, Wo, bo)


# speedup vs baseline: 1.6306x; 1.2083x over previous
import functools

import jax
import jax.numpy as jnp
from jax import lax
from jax.experimental import pallas as pl
from jax.experimental.pallas import tpu as pltpu
from jax.experimental.pallas import tpu_sc as plsc

N = 10000
D = 256
E = 160000
NCORE = 2
NSUB = 16
HALFN = 5120
EB = 64
NBT = 2520
CH = 24
TR = 5128
ZSR = HALFN // NSUB
OUTR = 2 * HALFN
DUMPDST = 4 * OUTR
BN = 1000


_INV_SQRT2 = 0.7071067811865476


def _gelu(h):
    return 0.5 * h * (1.0 + lax.erf(h * _INV_SQRT2))


def _mm0_body(x_ref, w_ref, b_ref, o_ref):
    r = jnp.dot(x_ref[...], w_ref[...], preferred_element_type=jnp.float32)
    o_ref[...] = r + b_ref[...]


def _fused_body(a_ref, g_ref, be_ref, w_ref, b_ref, o_ref):
    a = jnp.reshape(a_ref[...], (BN, D))
    mu = jnp.mean(a, axis=-1, keepdims=True)
    var = jnp.mean((a - mu) ** 2, axis=-1, keepdims=True)
    h = (a - mu) * lax.rsqrt(var + 1e-5) * g_ref[...] + be_ref[...]
    h = _gelu(h)
    r = jnp.dot(h, w_ref[...], preferred_element_type=jnp.float32)
    o_ref[...] = r + b_ref[...]


def _mm0(x, W, b):
    return pl.pallas_call(
        _mm0_body,
        grid=(N // BN,),
        in_specs=[
            pl.BlockSpec((BN, D), lambda i: (i, 0)),
            pl.BlockSpec((D, D), lambda i: (0, 0)),
            pl.BlockSpec((1, D), lambda i: (0, 0)),
        ],
        out_specs=pl.BlockSpec((BN, D), lambda i: (i, 0)),
        out_shape=jax.ShapeDtypeStruct((N, D), jnp.float32),
    )(x, W, b.reshape(1, D))


def _fused(a, g, be, W, b):
    return pl.pallas_call(
        _fused_body,
        grid=(N // BN,),
        in_specs=[
            pl.BlockSpec((2 * BN, D // 2), lambda i: (i, 0)),
            pl.BlockSpec((1, D), lambda i: (0, 0)),
            pl.BlockSpec((1, D), lambda i: (0, 0)),
            pl.BlockSpec((D, D), lambda i: (0, 0)),
            pl.BlockSpec((1, D), lambda i: (0, 0)),
        ],
        out_specs=pl.BlockSpec((BN, D), lambda i: (i, 0)),
        out_shape=jax.ShapeDtypeStruct((N, D), jnp.float32),
    )(a, g.reshape(1, D), be.reshape(1, D), W, b.reshape(1, D))


_SC_MESH = plsc.VectorSubcoreMesh(
    core_axis_name="c", subcore_axis_name="s", num_cores=NCORE, num_subcores=NSUB
)


@functools.partial(
    pl.kernel,
    out_type=jax.ShapeDtypeStruct((2 * OUTR, D // 2), jnp.float32),
    mesh=_SC_MESH,
    scratch_types=[
        pltpu.VMEM_SHARED((2 * TR, D // 2), jnp.float32),
        pltpu.VMEM((CH, EB), jnp.int32),
        pltpu.VMEM((CH, 2 * EB), jnp.int32),
        pltpu.VMEM((2, 2 * EB, D // 2), jnp.float32),
        pltpu.VMEM((NCORE, 16), jnp.int32),
        pltpu.SemaphoreType.DMA,
    ],
)
def _sc_scatter(hlin_hbm, srcg_hbm, dstg_hbm, zeros_hbm, ckb_hbm, out_hbm,
                table, src_v, dst_v, rows_v, bounds_v, gsem):
    cid = lax.axis_index("c")
    sid = lax.axis_index("s")

    pltpu.sync_copy(ckb_hbm, bounds_v)
    pltpu.sync_copy(zeros_hbm, table.at[pl.ds(sid * 2 * ZSR, 2 * ZSR)])
    brow = bounds_v[cid, pl.ds(0, 16)]
    klo = brow[0]
    khi = brow[1]
    per = (khi - klo + NSUB - 1) // NSUB
    lo_s = jnp.minimum(klo + sid * per, khi)
    hi_s = jnp.minimum(lo_s + per, khi)
    base = cid * (2 * HALFN)
    plsc.subcore_barrier()

    def chunk_body(k, carry):
        cb = k * CH
        pltpu.sync_copy(srcg_hbm.at[pl.ds(cb, CH)], src_v)
        pltpu.sync_copy(dstg_hbm.at[pl.ds(cb, CH)], dst_v)
        jlo = jnp.maximum(cb, lo_s)
        jhi = jnp.minimum(cb + CH, hi_s)

        def loc_body(i, c2):
            r = i - cb
            for c in range(2 * EB // 16):
                v = dst_v[r, pl.ds(c * 16, 16)]
                l = v - base
                oob = (l < 0) | (l >= 2 * HALFN)
                dst_v[r, pl.ds(c * 16, 16)] = jnp.where(oob, 2 * HALFN, l)
            return c2

        lax.fori_loop(jlo, jhi, loc_body, 0)

        @pl.when(jlo < jhi)
        def _():
            pltpu.async_copy(hlin_hbm.at[src_v.at[jlo - cb]],
                             rows_v.at[jlo % 2].reshape(EB, D), gsem)

        def b_body(j, c3):
            nxt = j + 1

            @pl.when(nxt < jhi)
            def _():
                pltpu.async_copy(hlin_hbm.at[src_v.at[nxt - cb]],
                                 rows_v.at[nxt % 2].reshape(EB, D), gsem)

            pltpu.make_async_copy(hlin_hbm.at[src_v.at[j - cb]],
                                  rows_v.at[j % 2].reshape(EB, D), gsem).wait()
            pltpu.sync_copy(rows_v.at[j % 2], table.at[dst_v.at[j - cb]],
                            add=True)
            return c3

        lax.fori_loop(jlo, jhi, b_body, 0)
        return carry

    lax.fori_loop(lo_s // CH, (hi_s + CH - 1) // CH, chunk_body, 0)
    plsc.subcore_barrier()

    pltpu.sync_copy(
        table.at[pl.ds(sid * 2 * ZSR, 2 * ZSR)],
        out_hbm.at[pl.ds(cid * 2 * HALFN + sid * 2 * ZSR, 2 * ZSR)])


def kernel(x, edge_index, W0, b0, g0, be0, W1, b1, g1, be1,
           W2, b2, g2, be2, W3, b3, g3, be3, Wo, bo):
    src = edge_index[0]
    dst = edge_index[1]
    half = (dst >= HALFN).astype(jnp.int32)
    perm = jnp.argsort(half)
    src = src[perm]
    dst = dst[perm]
    pad = NBT * EB - E
    srcg = jnp.concatenate(
        [src, jnp.zeros((pad,), jnp.int32)]).reshape(NBT, EB)
    dstp = jnp.concatenate([dst, jnp.full((pad,), DUMPDST, jnp.int32)])
    dstg = jnp.stack([2 * dstp, 2 * dstp + 1], axis=1).reshape(NBT, 2 * EB)
    split = jnp.sum(1 - half)
    hi0 = (split + EB - 1) // EB
    lo1 = split // EB
    z16 = jnp.zeros((NCORE,), jnp.int32)
    ckb = jnp.stack(
        [jnp.stack([jnp.int32(0), lo1]), jnp.stack([hi0, jnp.int32(NBT)])]
        + [z16] * 14, axis=1).astype(jnp.int32)
    zeros = jnp.zeros((2 * ZSR, D // 2), jnp.float32)

    hlin = _mm0(x, W0, b0)
    layers = [(g0, be0, W1, b1), (g1, be1, W2, b2), (g2, be2, W3, b3)]
    for (g, be, W, b) in layers:
        agg = _sc_scatter(hlin, srcg, dstg, zeros, ckb)
        hlin = _fused(agg, g, be, W, b)
    agg = _sc_scatter(hlin, srcg, dstg, zeros, ckb)
    return _fused(agg, g3, be3, Wo, bo)


# speedup vs baseline: 1.7127x; 1.0503x over previous
import functools

import jax
import jax.numpy as jnp
from jax import lax
from jax.experimental import pallas as pl
from jax.experimental.pallas import tpu as pltpu
from jax.experimental.pallas import tpu_sc as plsc

N = 10000
D = 256
E = 160000
NCORE = 2
NSUB = 16
HALFN = 5120
EB = 64
NBT = 2520
CH = 24
TR = 5128
ZSR = HALFN // NSUB
OUTR = 2 * HALFN
DUMPDST = 4 * OUTR
BN = 1000


_INV_SQRT2 = 0.7071067811865476


def _gelu(h):
    return 0.5 * h * (1.0 + lax.erf(h * _INV_SQRT2))


def _mm0_body(x_ref, w_ref, b_ref, o_ref):
    r = jnp.dot(x_ref[...], w_ref[...], preferred_element_type=jnp.float32)
    o_ref[...] = r + b_ref[...]


def _fused_body(a_ref, g_ref, be_ref, w_ref, b_ref, o_ref):
    a = jnp.reshape(a_ref[...], (BN, D))
    mu = jnp.mean(a, axis=-1, keepdims=True)
    var = jnp.mean((a - mu) ** 2, axis=-1, keepdims=True)
    h = (a - mu) * lax.rsqrt(var + 1e-5) * g_ref[...] + be_ref[...]
    h = _gelu(h)
    r = jnp.dot(h, w_ref[...], preferred_element_type=jnp.float32)
    o_ref[...] = r + b_ref[...]


def _mm0(x, W, b):
    return pl.pallas_call(
        _mm0_body,
        grid=(N // BN,),
        in_specs=[
            pl.BlockSpec((BN, D), lambda i: (i, 0)),
            pl.BlockSpec((D, D), lambda i: (0, 0)),
            pl.BlockSpec((1, D), lambda i: (0, 0)),
        ],
        out_specs=pl.BlockSpec((BN, D), lambda i: (i, 0)),
        out_shape=jax.ShapeDtypeStruct((N, D), jnp.float32),
    )(x, W, b.reshape(1, D))


def _fused(a, g, be, W, b):
    return pl.pallas_call(
        _fused_body,
        grid=(N // BN,),
        in_specs=[
            pl.BlockSpec((2 * BN, D // 2), lambda i: (i, 0)),
            pl.BlockSpec((1, D), lambda i: (0, 0)),
            pl.BlockSpec((1, D), lambda i: (0, 0)),
            pl.BlockSpec((D, D), lambda i: (0, 0)),
            pl.BlockSpec((1, D), lambda i: (0, 0)),
        ],
        out_specs=pl.BlockSpec((BN, D), lambda i: (i, 0)),
        out_shape=jax.ShapeDtypeStruct((N, D), jnp.float32),
    )(a, g.reshape(1, D), be.reshape(1, D), W, b.reshape(1, D))


_SC_MESH = plsc.VectorSubcoreMesh(
    core_axis_name="c", subcore_axis_name="s", num_cores=NCORE, num_subcores=NSUB
)


@functools.partial(
    pl.kernel,
    out_type=jax.ShapeDtypeStruct((2 * OUTR, D // 2), jnp.float32),
    mesh=_SC_MESH,
    scratch_types=[
        pltpu.VMEM_SHARED((2 * TR, D // 2), jnp.float32),
        pltpu.VMEM((CH, EB), jnp.int32),
        pltpu.VMEM((CH, 2 * EB), jnp.int32),
        pltpu.VMEM((2, 2 * EB, D // 2), jnp.float32),
        pltpu.VMEM((NCORE, 16), jnp.int32),
        pltpu.SemaphoreType.DMA,
    ],
)
def _sc_scatter(hlin_hbm, srcg_hbm, dstg_hbm, zeros_hbm, ckb_hbm, out_hbm,
                table, src_v, dst_v, rows_v, bounds_v, gsem):
    cid = lax.axis_index("c")
    sid = lax.axis_index("s")

    pltpu.sync_copy(ckb_hbm, bounds_v)
    pltpu.sync_copy(zeros_hbm, table.at[pl.ds(sid * 2 * ZSR, 2 * ZSR)])
    brow = bounds_v[cid, pl.ds(0, 16)]
    klo = brow[0]
    khi = brow[1]
    per = (khi - klo + NSUB - 1) // NSUB
    lo_s = jnp.minimum(klo + sid * per, khi)
    hi_s = jnp.minimum(lo_s + per, khi)
    base = cid * (2 * HALFN)
    plsc.subcore_barrier()

    def chunk_body(k, carry):
        cb = k * CH
        pltpu.sync_copy(srcg_hbm.at[pl.ds(cb, CH)], src_v)
        pltpu.sync_copy(dstg_hbm.at[pl.ds(cb, CH)], dst_v)
        jlo = jnp.maximum(cb, lo_s)
        jhi = jnp.minimum(cb + CH, hi_s)

        def loc_body(i, c2):
            r = i - cb
            for c in range(2 * EB // 16):
                v = dst_v[r, pl.ds(c * 16, 16)]
                l = v - base
                oob = (l < 0) | (l >= 2 * HALFN)
                dst_v[r, pl.ds(c * 16, 16)] = jnp.where(oob, 2 * HALFN, l)
            return c2

        lax.fori_loop(jlo, jhi, loc_body, 0)

        @pl.when(jlo < jhi)
        def _():
            pltpu.async_copy(hlin_hbm.at[src_v.at[jlo - cb]],
                             rows_v.at[jlo % 2].reshape(EB, D), gsem)

        def b_body(j, c3):
            nxt = j + 1

            @pl.when(nxt < jhi)
            def _():
                pltpu.async_copy(hlin_hbm.at[src_v.at[nxt - cb]],
                                 rows_v.at[nxt % 2].reshape(EB, D), gsem)

            pltpu.make_async_copy(hlin_hbm.at[src_v.at[j - cb]],
                                  rows_v.at[j % 2].reshape(EB, D), gsem).wait()
            pltpu.sync_copy(rows_v.at[j % 2], table.at[dst_v.at[j - cb]],
                            add=True)
            return c3

        lax.fori_loop(jlo, jhi, b_body, 0)
        return carry

    lax.fori_loop(lo_s // CH, (hi_s + CH - 1) // CH, chunk_body, 0)
    plsc.subcore_barrier()

    pltpu.sync_copy(
        table.at[pl.ds(sid * 2 * ZSR, 2 * ZSR)],
        out_hbm.at[pl.ds(cid * 2 * HALFN + sid * 2 * ZSR, 2 * ZSR)])


def kernel(x, edge_index, W0, b0, g0, be0, W1, b1, g1, be1,
           W2, b2, g2, be2, W3, b3, g3, be3, Wo, bo):
    src = edge_index[0]
    dst = edge_index[1]
    half = (dst >= HALFN).astype(jnp.int32)
    perm = jnp.argsort(half, stable=False)
    src = src[perm]
    dst = dst[perm]
    pad = NBT * EB - E
    srcg = jnp.concatenate(
        [src, jnp.zeros((pad,), jnp.int32)]).reshape(NBT, EB)
    dstp = jnp.concatenate([dst, jnp.full((pad,), DUMPDST, jnp.int32)])
    dstg = jnp.stack([2 * dstp, 2 * dstp + 1], axis=1).reshape(NBT, 2 * EB)
    split = jnp.sum(1 - half)
    hi0 = (split + EB - 1) // EB
    lo1 = split // EB
    z16 = jnp.zeros((NCORE,), jnp.int32)
    ckb = jnp.stack(
        [jnp.stack([jnp.int32(0), lo1]), jnp.stack([hi0, jnp.int32(NBT)])]
        + [z16] * 14, axis=1).astype(jnp.int32)
    zeros = jnp.zeros((2 * ZSR, D // 2), jnp.float32)

    hlin = _mm0(x, W0, b0)
    layers = [(g0, be0, W1, b1), (g1, be1, W2, b2), (g2, be2, W3, b3)]
    for (g, be, W, b) in layers:
        agg = _sc_scatter(hlin, srcg, dstg, zeros, ckb)
        hlin = _fused(agg, g, be, W, b)
    agg = _sc_scatter(hlin, srcg, dstg, zeros, ckb)
    return _fused(agg, g3, be3, Wo, bo)
